# Initial kernel scaffold; baseline (speedup 1.0000x reference)
import jax
import jax.numpy as jnp
from jax.experimental import pallas as pl


def kernel(input_ids, bigram_table, proj_w):
    raise NotImplementedError("write your pallas kernel here")


# speedup vs baseline: 14.2683x; 14.2683x over previous
import functools

import jax
import jax.numpy as jnp
from jax import lax
from jax.experimental import pallas as pl
from jax.experimental.pallas import tpu as pltpu
from jax.experimental.pallas import tpu_sc as plsc

N_BUCKETS = 1000000
BDIM = 32
DMODEL = 128
MULT = 1056


def _gather_body(ids_hbm, prev_hbm, table_hbm, emb_hbm, ids_v, prev_v, idx_v,
                 rows_v, sem, *, chunk, gb):
    nc = 2
    wid = lax.axis_index("s") * nc + lax.axis_index("c")
    base = wid * chunk

    pltpu.sync_copy(ids_hbm.at[pl.ds(base, chunk)], ids_v)
    pltpu.sync_copy(prev_hbm.at[pl.ds(base, chunk)], prev_v)

    def hash_body(i, carry):
        j0 = i * 16
        curr = ids_v[pl.ds(j0, 16)]
        prev = prev_v[pl.ds(j0, 16)]
        h = (prev * MULT + curr) % N_BUCKETS
        idx_v[pl.ds(j0, 16)] = h
        return carry

    lax.fori_loop(0, chunk // 16, hash_body, 0)

    ng = chunk // gb

    def gather_body(g, carry):
        idx_slice = idx_v.at[pl.ds(g * gb, gb)]
        pltpu.async_copy(table_hbm.at[idx_slice], rows_v, sem).wait()
        pltpu.sync_copy(rows_v, emb_hbm.at[pl.ds(base + g * gb, gb)])
        return carry

    lax.fori_loop(0, ng, gather_body, 0)


def _proj_body(x_ref, w_ref, o_ref):
    o_ref[...] = lax.dot_general(
        x_ref[...], w_ref[...],
        (((1,), (1,)), ((), ())),
        preferred_element_type=jnp.float32,
    )


@jax.jit
def kernel(input_ids, bigram_table, proj_w):
    batch, seq_len = input_ids.shape
    total = batch * seq_len
    nw = 32
    chunk = total // nw
    gb = 128

    ids_flat = input_ids.reshape(total)
    prev_flat = jnp.concatenate(
        [jnp.zeros((batch, 1), dtype=input_ids.dtype), input_ids[:, :-1]],
        axis=1,
    ).reshape(total)

    mesh = plsc.VectorSubcoreMesh(core_axis_name="c", subcore_axis_name="s")
    sc_gather = functools.partial(
        pl.kernel,
        mesh=mesh,
        out_type=jax.ShapeDtypeStruct((total, BDIM), jnp.float32),
        scratch_types=[
            pltpu.VMEM((chunk,), jnp.int32),
            pltpu.VMEM((chunk,), jnp.int32),
            pltpu.VMEM((chunk,), jnp.int32),
            pltpu.VMEM((gb, BDIM), jnp.float32),
            pltpu.SemaphoreType.DMA,
        ],
        compiler_params=pltpu.CompilerParams(use_tc_tiling_on_sc=False),
    )(functools.partial(_gather_body, chunk=chunk, gb=gb))

    emb = sc_gather(ids_flat, prev_flat, bigram_table)

    blk = 2048
    out = pl.pallas_call(
        _proj_body,
        grid=(total // blk,),
        in_specs=[
            pl.BlockSpec((blk, BDIM), lambda i: (i, 0)),
            pl.BlockSpec((DMODEL, BDIM), lambda i: (0, 0)),
        ],
        out_specs=pl.BlockSpec((blk, DMODEL), lambda i: (i, 0)),
        out_shape=jax.ShapeDtypeStruct((total, DMODEL), jnp.float32),
    )(emb, proj_w)

    return out.reshape(batch, seq_len, DMODEL)


# speedup vs baseline: 18.2805x; 1.2812x over previous
import functools

import jax
import jax.numpy as jnp
from jax import lax
from jax.experimental import pallas as pl
from jax.experimental.pallas import tpu as pltpu
from jax.experimental.pallas import tpu_sc as plsc

N_BUCKETS = 1000000
BDIM = 32
DMODEL = 128
MULT = 1056
GUARD = 8


def _gather_body(ids_hbm, table_hbm, emb_hbm, ids_v, idx_v, rows_v, sem,
                 *, total, chunk, seq_len, gb):
    nc = 2
    wid = lax.axis_index("s") * nc + lax.axis_index("c")
    base = wid * chunk

    zeros16 = jnp.zeros((16,), jnp.int32)
    ids_v[pl.ds(0, 16)] = zeros16
    pltpu.sync_copy(ids_hbm.at[pl.ds(base, chunk)], ids_v.at[pl.ds(GUARD, chunk)])

    lanes = lax.iota(jnp.int32, 16)

    def hash_body(i, carry):
        j0 = i * 16
        curr = ids_v[pl.ds(j0 + GUARD, 16)]
        prev = ids_v[pl.ds(j0 + GUARD - 1, 16)]
        col = (lanes + j0) % seq_len
        prev = jnp.where(col == 0, 0, prev)
        h = (prev * MULT + curr) % N_BUCKETS
        idx_v[pl.ds(j0, 16)] = h
        return carry

    lax.fori_loop(0, chunk // 16, hash_body, 0)

    ng = chunk // gb

    def gather_body(g, carry):
        idx_slice = idx_v.at[pl.ds(g * gb, gb)]
        pltpu.async_copy(table_hbm.at[idx_slice], rows_v, sem).wait()
        pltpu.sync_copy(rows_v, emb_hbm.at[pl.ds(base + g * gb, gb)])
        return carry

    lax.fori_loop(0, ng, gather_body, 0)


def _proj_body(x_ref, w_ref, o_ref, *, blk):
    y = lax.dot_general(
        x_ref[...], w_ref[...],
        (((1,), (0,)), ((), ())),
        preferred_element_type=jnp.float32,
    )
    o_ref[...] = y.reshape(4 * blk, DMODEL)


@jax.jit
def kernel(input_ids, bigram_table, proj_w):
    batch, seq_len = input_ids.shape
    total = batch * seq_len
    total4 = total // 4
    nw = 32
    chunk = total // nw
    gb = 128

    ids_flat = input_ids.reshape(total)

    mesh = plsc.VectorSubcoreMesh(core_axis_name="c", subcore_axis_name="s")
    sc_gather = functools.partial(
        pl.kernel,
        mesh=mesh,
        out_type=jax.ShapeDtypeStruct((total, BDIM), jnp.float32),
        scratch_types=[
            pltpu.VMEM((chunk + GUARD,), jnp.int32),
            pltpu.VMEM((chunk,), jnp.int32),
            pltpu.VMEM((gb, BDIM), jnp.float32),
            pltpu.SemaphoreType.DMA,
        ],
        compiler_params=pltpu.CompilerParams(use_tc_tiling_on_sc=False),
    )(functools.partial(_gather_body, total=total, chunk=chunk,
                        seq_len=seq_len, gb=gb))

    emb4 = sc_gather(ids_flat, bigram_table).reshape(total4, DMODEL)

    wt = proj_w.T
    w4 = jnp.zeros((DMODEL, 4 * DMODEL), jnp.float32)
    for k in range(4):
        w4 = lax.dynamic_update_slice(w4, wt, (BDIM * k, DMODEL * k))

    blk = 512
    out = pl.pallas_call(
        functools.partial(_proj_body, blk=blk),
        grid=(total4 // blk,),
        in_specs=[
            pl.BlockSpec((blk, DMODEL), lambda i: (i, 0)),
            pl.BlockSpec((DMODEL, 4 * DMODEL), lambda i: (0, 0)),
        ],
        out_specs=pl.BlockSpec((4 * blk, DMODEL), lambda i: (i, 0)),
        out_shape=jax.ShapeDtypeStruct((total, DMODEL), jnp.float32),
    )(emb4, w4)

    return out.reshape(batch, seq_len, DMODEL)


# speedup vs baseline: 20.9874x; 1.1481x over previous
import functools

import jax
import jax.numpy as jnp
from jax import lax
from jax.experimental import pallas as pl
from jax.experimental.pallas import tpu as pltpu
from jax.experimental.pallas import tpu_sc as plsc

N_BUCKETS = 1000000
BDIM = 32
DMODEL = 128
MULT = 1056
GUARD = 8


def _gather_body(ids_hbm, table_hbm, emb_hbm, ids_v, idx_v, rows_a, rows_b,
                 gsem_a, gsem_b, *, total, chunk, seq_len, gb, kf):
    nc = 2
    wid = lax.axis_index("s") * nc + lax.axis_index("c")
    base = wid * chunk
    bigb = gb * kf
    nbig = chunk // bigb

    zeros16 = jnp.zeros((16,), jnp.int32)
    ids_v[pl.ds(0, 16)] = zeros16
    pltpu.sync_copy(ids_hbm.at[pl.ds(base, chunk)], ids_v.at[pl.ds(GUARD, chunk)])

    lanes = lax.iota(jnp.int32, 16)

    def hash_body(i, carry):
        j0 = i * 16
        curr = ids_v[pl.ds(j0 + GUARD, 16)]
        prev = ids_v[pl.ds(j0 + GUARD - 1, 16)]
        col = (lanes + j0) % seq_len
        prev = jnp.where(col == 0, 0, prev)
        h = (prev * MULT + curr) % N_BUCKETS
        idx_v[pl.ds(j0, 16)] = h
        return carry

    lax.fori_loop(0, chunk // 16, hash_body, 0)

    def fire(buf, gsem, c):
        for k in range(kf):
            idx_slice = idx_v.at[pl.ds(c * bigb + k * gb, gb)]
            pltpu.async_copy(
                table_hbm.at[idx_slice], buf.at[pl.ds(k * gb, gb)], gsem
            )

    def drain(buf, gsem):
        pltpu.make_async_copy(emb_hbm.at[pl.ds(0, bigb)], buf, gsem).wait()

    def out_copy(buf, c):
        pltpu.sync_copy(buf, emb_hbm.at[pl.ds(base + c * bigb, bigb)])

    fire(rows_a, gsem_a, 0)

    def pipe_body(p, carry):
        ca = 2 * p
        fire(rows_b, gsem_b, ca + 1)
        drain(rows_a, gsem_a)
        out_copy(rows_a, ca)
        fire(rows_a, gsem_a, ca + 2)
        drain(rows_b, gsem_b)
        out_copy(rows_b, ca + 1)
        return carry

    lax.fori_loop(0, (nbig - 1) // 2, pipe_body, 0)

    drain(rows_a, gsem_a)
    out_copy(rows_a, nbig - 1)


def _proj_body(x_ref, w_ref, o_ref, *, blk):
    y = lax.dot_general(
        x_ref[...], w_ref[...],
        (((1,), (0,)), ((), ())),
        preferred_element_type=jnp.float32,
    )
    o_ref[...] = y.reshape(4 * blk, DMODEL)


@jax.jit
def kernel(input_ids, bigram_table, proj_w):
    batch, seq_len = input_ids.shape
    total = batch * seq_len
    total4 = total // 4
    nw = 32
    chunk = total // nw
    gb = 128
    kf = 8

    ids_flat = input_ids.reshape(total)

    mesh = plsc.VectorSubcoreMesh(core_axis_name="c", subcore_axis_name="s")
    sc_gather = functools.partial(
        pl.kernel,
        mesh=mesh,
        out_type=jax.ShapeDtypeStruct((total, BDIM), jnp.float32),
        scratch_types=[
            pltpu.VMEM((chunk + GUARD,), jnp.int32),
            pltpu.VMEM((chunk,), jnp.int32),
            pltpu.VMEM((gb * kf, BDIM), jnp.float32),
            pltpu.VMEM((gb * kf, BDIM), jnp.float32),
            pltpu.SemaphoreType.DMA,
            pltpu.SemaphoreType.DMA,
        ],
        compiler_params=pltpu.CompilerParams(use_tc_tiling_on_sc=False),
    )(functools.partial(_gather_body, total=total, chunk=chunk,
                        seq_len=seq_len, gb=gb, kf=kf))

    emb4 = sc_gather(ids_flat, bigram_table).reshape(total4, DMODEL)

    wt = proj_w.T
    w4 = jnp.zeros((DMODEL, 4 * DMODEL), jnp.float32)
    for k in range(4):
        w4 = lax.dynamic_update_slice(w4, wt, (BDIM * k, DMODEL * k))

    blk = 512
    out = pl.pallas_call(
        functools.partial(_proj_body, blk=blk),
        grid=(total4 // blk,),
        in_specs=[
            pl.BlockSpec((blk, DMODEL), lambda i: (i, 0)),
            pl.BlockSpec((DMODEL, 4 * DMODEL), lambda i: (0, 0)),
        ],
        out_specs=pl.BlockSpec((4 * blk, DMODEL), lambda i: (i, 0)),
        out_shape=jax.ShapeDtypeStruct((total, DMODEL), jnp.float32),
    )(emb4, w4)

    return out.reshape(batch, seq_len, DMODEL)
